# baseline (speedup 1.0000x reference)
import functools

import jax
import jax.numpy as jnp
from jax import lax
from jax.experimental import pallas as pl
from jax.experimental.pallas import tpu as pltpu
from jax.experimental.pallas import tpu_sc as plsc

B = 64
C = 768
HW = 784
K = 256
NCORE = 2
NSUB = 16
NW = NCORE * NSUB
BPW = B // NW
CH = 64
L = 16
NCOL = HW // L
GCH = 8
NGRP = C // (GCH * L)


def _sc_body(x_hbm, out_hbm, buf, scores, idx, sem):
    cid = lax.axis_index("c")
    sid = lax.axis_index("s")
    wid = sid * NCORE + cid
    lane = lax.iota(jnp.int32, L)

    def _batch(bi, _):
        b_glob = wid * BPW + bi
        row0 = b_glob * C

        def _chunk1(ch, _):
            pltpu.sync_copy(
                x_hbm.at[pl.ds((row0 + ch * CH) * HW, CH * HW)], buf)

            def _row(r, _):
                base = r * HW
                accs = [jnp.zeros((L,), jnp.float32) for _ in range(4)]
                for jj in range(NCOL):
                    v = buf[pl.ds(base + jj * L, L)]
                    accs[jj % 4] = jnp.maximum(accs[jj % 4], jnp.abs(v))
                acc = jnp.maximum(jnp.maximum(accs[0], accs[1]),
                                  jnp.maximum(accs[2], accs[3]))
                sub = r % L
                gbase = ch * CH + (r // L) * L
                vec = scores[pl.ds(gbase, L)]
                scores[pl.ds(gbase, L)] = jnp.where(lane == sub,
                                                    jnp.max(acc), vec)
                return 0

            lax.fori_loop(0, CH, _row, 0)
            return 0

        lax.fori_loop(0, C // CH, _chunk1, 0)

        def _group(g, _):
            subj = [scores[pl.ds((g * GCH + i) * L, L)] for i in range(GCH)]

            def _mk_seg(cmp_ge):
                def _seg(jv, accs, _ge=cmp_ge):
                    v = scores[pl.ds(jv * L, L)]
                    out = []
                    for i in range(GCH):
                        cnt = jnp.zeros((L,), jnp.int32)
                        for sub in range(L):
                            bc = jnp.full((L,), v[sub])
                            cc = bc >= subj[i] if _ge else bc > subj[i]
                            cnt = cnt + cc.astype(jnp.int32)
                        out.append(accs[i] + cnt)
                    return tuple(out)
                return _seg

            def _mid(jv, accs):
                v = scores[pl.ds(jv * L, L)]
                out = list(accs)
                for i in range(GCH):
                    idxv = (g * GCH + i) * L + lane
                    cnt = jnp.zeros((L,), jnp.int32)
                    for sub in range(L):
                        j = jv * L + sub
                        bc = jnp.full((L,), v[sub])
                        sel = jnp.where(j < idxv, bc >= subj[i],
                                        bc > subj[i])
                        cnt = cnt + sel.astype(jnp.int32)
                    out[i] = out[i] + cnt
                return tuple(out)

            zeros = tuple(jnp.zeros((L,), jnp.int32) for _ in range(GCH))
            accs = lax.fori_loop(0, g * GCH, _mk_seg(True), zeros)
            accs = lax.fori_loop((g + 1) * GCH, C // L, _mk_seg(False),
                                 accs)
            accs = lax.fori_loop(g * GCH, (g + 1) * GCH, _mid, accs)

            for i in range(GCH):
                rank = accs[i]
                chan = row0 + (g * GCH + i) * L + lane
                m = rank < K
                plsc.store_scatter(idx, [jnp.where(m, rank, 0)], chan,
                                   mask=m)
            return 0

        lax.fori_loop(0, NGRP, _group, 0)

        def _g3(g3, _):
            def _grp(grp, _):
                idx_v = idx[pl.ds(g3 * CH + grp * L, L)]
                for sub in range(L):
                    pltpu.make_async_copy(
                        x_hbm.at[pl.ds(idx_v[sub] * HW, HW)],
                        buf.at[pl.ds((grp * L + sub) * HW, HW)],
                        sem).start()
                return 0

            lax.fori_loop(0, CH // L, _grp, 0)

            def _drain(grp, _):
                pltpu.make_async_copy(
                    x_hbm.at[pl.ds(0, HW)],
                    buf.at[pl.ds(grp * HW, HW)], sem).wait()
                return 0

            lax.fori_loop(0, CH, _drain, 0)
            pltpu.sync_copy(
                buf, out_hbm.at[pl.ds((b_glob * K + g3 * CH) * HW,
                                      CH * HW)])
            return 0

        lax.fori_loop(0, K // CH, _g3, 0)
        return 0

    lax.fori_loop(0, BPW, _batch, 0)


_sc_kernel = functools.partial(
    pl.kernel,
    mesh=plsc.VectorSubcoreMesh(core_axis_name="c", subcore_axis_name="s"),
    compiler_params=pltpu.CompilerParams(needs_layout_passes=False),
    out_type=jax.ShapeDtypeStruct((B * K * HW,), jnp.float32),
    scratch_types=[
        pltpu.VMEM((CH * HW,), jnp.float32),
        pltpu.VMEM((C,), jnp.float32),
        pltpu.VMEM((K,), jnp.int32),
        pltpu.SemaphoreType.DMA,
    ],
)(_sc_body)


def kernel(x):
    b, c, h, w = x.shape
    x1 = x.reshape(b * c * h * w)
    out1 = _sc_kernel(x1)
    return out1.reshape(b, K, h, w)


# speedup vs baseline: 1.5927x; 1.5927x over previous
import functools

import jax
import jax.numpy as jnp
from jax import lax
from jax.experimental import pallas as pl
from jax.experimental.pallas import tpu as pltpu
from jax.experimental.pallas import tpu_sc as plsc

B = 64
C = 768
HW = 784
K = 256
NCORE = 2
NSUB = 16
NW = NCORE * NSUB
BPW = B // NW
CH = 64
L = 16
NCOL = HW // L
GCH = 8
NGRP = C // (GCH * L)


def _sc_body(x_hbm, out_hbm, buf, scores, idx, sem):
    cid = lax.axis_index("c")
    sid = lax.axis_index("s")
    wid = sid * NCORE + cid
    lane = lax.iota(jnp.int32, L)

    def _batch(bi, _):
        b_glob = wid * BPW + bi
        row0 = b_glob * C

        def _chunk1(ch, _):
            pltpu.sync_copy(x_hbm.at[pl.ds(row0 + ch * CH, CH)], buf)

            def _row(r, _):
                accs = [jnp.zeros((L,), jnp.float32) for _ in range(4)]
                for jj in range(NCOL):
                    v = buf[r, pl.ds(jj * L, L)]
                    accs[jj % 4] = jnp.maximum(accs[jj % 4], jnp.abs(v))
                acc = jnp.maximum(jnp.maximum(accs[0], accs[1]),
                                  jnp.maximum(accs[2], accs[3]))
                sub = r % L
                gbase = ch * CH + (r // L) * L
                vec = scores[pl.ds(gbase, L)]
                scores[pl.ds(gbase, L)] = jnp.where(lane == sub,
                                                    jnp.max(acc), vec)
                return 0

            lax.fori_loop(0, CH, _row, 0)
            return 0

        lax.fori_loop(0, C // CH, _chunk1, 0)

        def _group(g, _):
            subj = [scores[pl.ds((g * GCH + i) * L, L)] for i in range(GCH)]

            def _mk_seg(cmp_ge):
                def _seg(jv, accs, _ge=cmp_ge):
                    v = scores[pl.ds(jv * L, L)]
                    out = []
                    for i in range(GCH):
                        cnt = jnp.zeros((L,), jnp.int32)
                        for sub in range(L):
                            bc = jnp.full((L,), v[sub])
                            cc = bc >= subj[i] if _ge else bc > subj[i]
                            cnt = cnt + cc.astype(jnp.int32)
                        out.append(accs[i] + cnt)
                    return tuple(out)
                return _seg

            def _mid(jv, accs):
                v = scores[pl.ds(jv * L, L)]
                out = list(accs)
                for i in range(GCH):
                    idxv = (g * GCH + i) * L + lane
                    cnt = jnp.zeros((L,), jnp.int32)
                    for sub in range(L):
                        j = jv * L + sub
                        bc = jnp.full((L,), v[sub])
                        sel = jnp.where(j < idxv, bc >= subj[i],
                                        bc > subj[i])
                        cnt = cnt + sel.astype(jnp.int32)
                    out[i] = out[i] + cnt
                return tuple(out)

            zeros = tuple(jnp.zeros((L,), jnp.int32) for _ in range(GCH))
            accs = lax.fori_loop(0, g * GCH, _mk_seg(True), zeros)
            accs = lax.fori_loop((g + 1) * GCH, C // L, _mk_seg(False),
                                 accs)
            accs = lax.fori_loop(g * GCH, (g + 1) * GCH, _mid, accs)

            for i in range(GCH):
                rank = accs[i]
                chan = row0 + (g * GCH + i) * L + lane
                m = rank < K
                plsc.store_scatter(idx, [jnp.where(m, rank, 0)], chan,
                                   mask=m)
            return 0

        lax.fori_loop(0, NGRP, _group, 0)

        def _g3(g3, _):
            def _grp(grp, _):
                idx_v = idx[pl.ds(g3 * CH + grp * L, L)]
                for sub in range(L):
                    pltpu.make_async_copy(
                        x_hbm.at[idx_v[sub]],
                        buf.at[grp * L + sub], sem).start()
                return 0

            lax.fori_loop(0, CH // L, _grp, 0)

            def _drain(grp, _):
                pltpu.make_async_copy(
                    x_hbm.at[0], buf.at[grp], sem).wait()
                return 0

            lax.fori_loop(0, CH, _drain, 0)
            pltpu.sync_copy(
                buf, out_hbm.at[pl.ds(b_glob * K + g3 * CH, CH)])
            return 0

        lax.fori_loop(0, K // CH, _g3, 0)
        return 0

    lax.fori_loop(0, BPW, _batch, 0)


_sc_kernel = functools.partial(
    pl.kernel,
    mesh=plsc.VectorSubcoreMesh(core_axis_name="c", subcore_axis_name="s"),
    compiler_params=pltpu.CompilerParams(needs_layout_passes=False,
                                         use_tc_tiling_on_sc=True),
    out_type=jax.ShapeDtypeStruct((B * K, HW), jnp.float32),
    scratch_types=[
        pltpu.VMEM((CH, HW), jnp.float32),
        pltpu.VMEM((C,), jnp.float32),
        pltpu.VMEM((K,), jnp.int32),
        pltpu.SemaphoreType.DMA,
    ],
)(_sc_body)


def kernel(x):
    b, c, h, w = x.shape
    x2 = x.reshape(b * c, h * w)
    out2 = _sc_kernel(x2)
    return out2.reshape(b, K, h, w)


# speedup vs baseline: 1.6701x; 1.0486x over previous
import functools

import jax
import jax.numpy as jnp
from jax import lax
from jax.experimental import pallas as pl
from jax.experimental.pallas import tpu as pltpu
from jax.experimental.pallas import tpu_sc as plsc

B = 64
C = 768
HW = 784
K = 256
NCORE = 2
NSUB = 16
NW = NCORE * NSUB
BPW = B // NW
CH = 32
L = 16
NCOL = HW // L
NCHUNK = C // CH
NG3 = K // CH
GCH = 8
NGRP = C // (GCH * L)


def _sc_body(x_hbm, out_hbm, bufa, bufb, scores, idx, sem):
    cid = lax.axis_index("c")
    sid = lax.axis_index("s")
    wid = sid * NCORE + cid
    lane = lax.iota(jnp.int32, L)

    def _batch(bi, _):
        b_glob = wid * BPW + bi
        row0 = b_glob * C

        def _start1(c, dst):
            pltpu.make_async_copy(
                x_hbm.at[pl.ds(row0 + c * CH, CH)], dst, sem).start()

        def _wait1(dst):
            pltpu.make_async_copy(
                x_hbm.at[pl.ds(0, CH)], dst, sem).wait()

        def _compute1(c, src):
            def _row(r, _):
                accs = [jnp.zeros((L,), jnp.float32) for _ in range(4)]
                for jj in range(NCOL):
                    v = src[r, pl.ds(jj * L, L)]
                    accs[jj % 4] = jnp.maximum(accs[jj % 4], jnp.abs(v))
                acc = jnp.maximum(jnp.maximum(accs[0], accs[1]),
                                  jnp.maximum(accs[2], accs[3]))
                sub = r % L
                gbase = c * CH + (r // L) * L
                vec = scores[pl.ds(gbase, L)]
                scores[pl.ds(gbase, L)] = jnp.where(lane == sub,
                                                    jnp.max(acc), vec)
                return 0

            lax.fori_loop(0, CH, _row, 0)

        _start1(0, bufa)

        def _pair1(p, _):
            c0 = 2 * p
            _wait1(bufa)
            _start1(c0 + 1, bufb)
            _compute1(c0, bufa)
            _wait1(bufb)
            _start1((c0 + 2) % NCHUNK, bufa)
            _compute1(c0 + 1, bufb)
            return 0

        lax.fori_loop(0, NCHUNK // 2, _pair1, 0)
        _wait1(bufa)

        def _group(g, _):
            subj = [scores[pl.ds((g * GCH + i) * L, L)] for i in range(GCH)]

            def _mk_seg(cmp_ge):
                def _seg(jv, accs, _ge=cmp_ge):
                    v = scores[pl.ds(jv * L, L)]
                    out = []
                    for i in range(GCH):
                        cnt = jnp.zeros((L,), jnp.int32)
                        for sub in range(L):
                            bc = jnp.full((L,), v[sub])
                            cc = bc >= subj[i] if _ge else bc > subj[i]
                            cnt = cnt + cc.astype(jnp.int32)
                        out.append(accs[i] + cnt)
                    return tuple(out)
                return _seg

            def _mid(jv, accs):
                v = scores[pl.ds(jv * L, L)]
                out = list(accs)
                for i in range(GCH):
                    idxv = (g * GCH + i) * L + lane
                    cnt = jnp.zeros((L,), jnp.int32)
                    for sub in range(L):
                        j = jv * L + sub
                        bc = jnp.full((L,), v[sub])
                        sel = jnp.where(j < idxv, bc >= subj[i],
                                        bc > subj[i])
                        cnt = cnt + sel.astype(jnp.int32)
                    out[i] = out[i] + cnt
                return tuple(out)

            zeros = tuple(jnp.zeros((L,), jnp.int32) for _ in range(GCH))
            accs = lax.fori_loop(0, g * GCH, _mk_seg(True), zeros)
            accs = lax.fori_loop((g + 1) * GCH, C // L, _mk_seg(False),
                                 accs)
            accs = lax.fori_loop(g * GCH, (g + 1) * GCH, _mid, accs)

            for i in range(GCH):
                rank = accs[i]
                chan = row0 + (g * GCH + i) * L + lane
                m = rank < K
                plsc.store_scatter(idx, [jnp.where(m, rank, 0)], chan,
                                   mask=m)
            return 0

        lax.fori_loop(0, NGRP, _group, 0)

        def _fetch3(g3, dst):
            def _grp(grp, _):
                idx_v = idx[pl.ds(g3 * CH + grp * L, L)]
                for sub in range(L):
                    pltpu.make_async_copy(
                        x_hbm.at[idx_v[sub]],
                        dst.at[grp * L + sub], sem).start()
                return 0

            lax.fori_loop(0, CH // L, _grp, 0)

        def _drain3(dst):
            def _d(grp, _):
                pltpu.make_async_copy(
                    x_hbm.at[0], dst.at[grp], sem).wait()
                return 0

            lax.fori_loop(0, CH, _d, 0)

        def _write3(g3, src):
            pltpu.sync_copy(
                src, out_hbm.at[pl.ds(b_glob * K + g3 * CH, CH)])

        _fetch3(0, bufa)

        def _pair3(p, _):
            g0 = 2 * p
            _drain3(bufa)
            _fetch3(g0 + 1, bufb)
            _write3(g0, bufa)
            _drain3(bufb)
            _fetch3((g0 + 2) % NG3, bufa)
            _write3(g0 + 1, bufb)
            return 0

        lax.fori_loop(0, NG3 // 2, _pair3, 0)
        _drain3(bufa)
        return 0

    lax.fori_loop(0, BPW, _batch, 0)


_sc_kernel = functools.partial(
    pl.kernel,
    mesh=plsc.VectorSubcoreMesh(core_axis_name="c", subcore_axis_name="s"),
    compiler_params=pltpu.CompilerParams(needs_layout_passes=False,
                                         use_tc_tiling_on_sc=True),
    out_type=jax.ShapeDtypeStruct((B * K, HW), jnp.float32),
    scratch_types=[
        pltpu.VMEM((CH, HW), jnp.float32),
        pltpu.VMEM((CH, HW), jnp.float32),
        pltpu.VMEM((C,), jnp.float32),
        pltpu.VMEM((K,), jnp.int32),
        pltpu.SemaphoreType.DMA,
    ],
)(_sc_body)


def kernel(x):
    b, c, h, w = x.shape
    x2 = x.reshape(b * c, h * w)
    out2 = _sc_kernel(x2)
    return out2.reshape(b, K, h, w)
